# baseline (speedup 1.0000x reference)
import functools

import jax
import jax.numpy as jnp
from jax.experimental import pallas as pl
from jax.experimental.pallas import tpu as pltpu

E = 8
K = 2
D = 1024
H = 1024
B = 2048

BM = 256
NEG = -1e30


def _gating_body(x_ref, gw_ref, gb_ref, probs_ref, tki_ref, dgt_ref):
    x = x_ref[...]
    gw = gw_ref[...]
    logits = jnp.dot(x, gw, preferred_element_type=jnp.float32) + gb_ref[...][None, :]
    m = jnp.max(logits, axis=1, keepdims=True)
    ex = jnp.exp(logits - m)
    probs_ref[...] = ex / jnp.sum(ex, axis=1, keepdims=True)
    idx = jax.lax.broadcasted_iota(jnp.int32, (B, E), 1)
    m1 = jnp.max(logits, axis=1, keepdims=True)
    i1 = jnp.min(jnp.where(logits == m1, idx, E), axis=1, keepdims=True)
    masked = jnp.where(idx == i1, NEG, logits)
    m2 = jnp.max(masked, axis=1, keepdims=True)
    i2 = jnp.min(jnp.where(masked == m2, idx, E), axis=1, keepdims=True)
    tki_ref[...] = jnp.concatenate([i1, i2], axis=1)
    b = jnp.exp(m2 - m1)
    g1 = 1.0 / (1.0 + b)
    g2 = b / (1.0 + b)
    dg = g1 * (idx == i1) + g2 * (idx == i2)
    dgt_ref[...] = dg.T


def _ffn_body(x_ref, w1_ref, b1_ref, w2_ref, b2_ref, dgt_ref, out_ref):
    e = pl.program_id(0)
    bt = pl.program_id(1)
    xt = x_ref[...].astype(jnp.bfloat16)
    w1 = w1_ref[...].astype(jnp.bfloat16)
    h = jnp.dot(xt, w1, preferred_element_type=jnp.float32) + b1_ref[0][None, :]
    h = jnp.maximum(h, 0.0).astype(jnp.bfloat16)
    w2 = w2_ref[...].astype(jnp.bfloat16)
    eo = jnp.dot(h, w2, preferred_element_type=jnp.float32) + b2_ref[0][None, :]
    contrib = eo * dgt_ref[0][:, None]
    rows = pl.ds(bt * BM, BM)

    @pl.when(e == 0)
    def _():
        out_ref[rows, :] = contrib

    @pl.when(e != 0)
    def _():
        out_ref[rows, :] += contrib


@jax.jit
def _moe(x, gate_W, gate_b, W1, b1, W2, b2):
    probs, tki, dgt = pl.pallas_call(
        _gating_body,
        out_shape=(
            jax.ShapeDtypeStruct((B, E), jnp.float32),
            jax.ShapeDtypeStruct((B, K), jnp.int32),
            jax.ShapeDtypeStruct((E, B), jnp.float32),
        ),
    )(x, gate_W, gate_b)

    out = pl.pallas_call(
        _ffn_body,
        grid=(E, B // BM),
        in_specs=[
            pl.BlockSpec((BM, D), lambda e, bt: (bt, 0)),
            pl.BlockSpec((None, D, H), lambda e, bt: (e, 0, 0)),
            pl.BlockSpec((None, 1, H), lambda e, bt: (e, 0, 0)),
            pl.BlockSpec((None, H, D), lambda e, bt: (e, 0, 0)),
            pl.BlockSpec((None, 1, D), lambda e, bt: (e, 0, 0)),
            pl.BlockSpec((None, 1, BM), lambda e, bt: (e, 0, bt)),
        ],
        out_specs=pl.BlockSpec((B, D), lambda e, bt: (0, 0)),
        out_shape=jax.ShapeDtypeStruct((B, D), jnp.float32),
    )(x, W1, b1[:, None, :], W2, b2[:, None, :], dgt[:, None, :])
    return out, probs, tki


def kernel(x, gate_W, gate_b, W1, b1, W2, b2):
    return _moe(x, gate_W, gate_b, W1, b1, W2, b2)


# speedup vs baseline: 1.0121x; 1.0121x over previous
import functools

import jax
import jax.numpy as jnp
from jax import lax
from jax.experimental import pallas as pl
from jax.experimental.pallas import tpu as pltpu
from jax.experimental.pallas import tpu_sc as plsc

E = 8
K = 2
D = 1024
H = 1024
B = 2048

T = 256
MT = 24
P = MT * T
NC = 2
NS = 16
NW = NC * NS
TPW = B // NW
NEG = -1e30


def _gating_body(x_ref, gw_ref, gb_ref, probs_ref, tki_ref, pos_ref, g_ref,
                 g16_ref, meta_ref):
    x = x_ref[...]
    logits = jnp.dot(x, gw_ref[...], preferred_element_type=jnp.float32)
    logits = logits + gb_ref[...][None, :]
    m = jnp.max(logits, axis=1, keepdims=True)
    ex = jnp.exp(logits - m)
    probs_ref[...] = ex / jnp.sum(ex, axis=1, keepdims=True)

    idx = lax.broadcasted_iota(jnp.int32, (B, E), 1)
    i1 = jnp.min(jnp.where(logits == m, idx, E), axis=1, keepdims=True)
    oh1 = idx == i1
    masked = jnp.where(oh1, NEG, logits)
    m2 = jnp.max(masked, axis=1, keepdims=True)
    i2 = jnp.min(jnp.where(masked == m2, idx, E), axis=1, keepdims=True)
    oh2 = idx == i2
    tki_ref[...] = jnp.concatenate([i1, i2], axis=1)

    b = jnp.exp(m2 - m)
    g1 = 1.0 / (1.0 + b)
    g2 = b / (1.0 + b)
    g_ref[...] = jnp.concatenate([g1, g2], axis=1)
    ones16 = jnp.ones((1, 128), jnp.float32)
    g16_ref[...] = jnp.concatenate([g1 * ones16, g2 * ones16], axis=1)

    cnt = oh1.astype(jnp.int32) + oh2.astype(jnp.int32)
    c = cnt
    s = 1
    while s < B:
        c = c + jnp.concatenate(
            [jnp.zeros((s, E), jnp.int32), c[:-s]], axis=0)
        s *= 2
    counts = c[B - 1:B, :]
    tiles_e = (counts + (T - 1)) // T
    tin = tiles_e
    s = 1
    while s < E:
        tin = tin + jnp.concatenate(
            [jnp.zeros((1, s), jnp.int32), tin[:, :-s]], axis=1)
        s *= 2
    toff = tin - tiles_e
    num_tiles = tin[:, E - 1:E]

    rank1 = jnp.sum(c * oh1, axis=1, keepdims=True) - 1
    rank2 = jnp.sum(c * oh2, axis=1, keepdims=True) - 1
    base1 = jnp.sum(jnp.broadcast_to(toff, (B, E)) * oh1, axis=1,
                    keepdims=True) * T
    base2 = jnp.sum(jnp.broadcast_to(toff, (B, E)) * oh2, axis=1,
                    keepdims=True) * T
    pos_ref[...] = jnp.concatenate([base1 + rank1, base2 + rank2], axis=1)

    miota = lax.broadcasted_iota(jnp.int32, (1, 128), 1)
    eot = jnp.zeros((1, 128), jnp.int32)
    for e in range(E):
        eot = eot + (miota >= toff[0, e]).astype(jnp.int32)
    eot = eot - 1
    real = (miota < num_tiles).astype(jnp.int32)
    xidx = jnp.minimum(miota, num_tiles - 1)
    meta_ref[...] = jnp.concatenate(
        [eot, real, xidx, jnp.zeros((1, 128), jnp.int32)], axis=0)


def _dispatch_body(x_hbm, pos_hbm, g16_hbm, xs_hbm, gs_hbm,
                   xv, idxv, g16v, sem):
    wid = lax.axis_index("s") * NC + lax.axis_index("c")
    base = wid * TPW
    pltpu.sync_copy(x_hbm.at[pl.ds(base, TPW)], xv)
    for k in range(K):
        pltpu.sync_copy(pos_hbm.at[k, wid], idxv)
        pltpu.async_copy(xv, xs_hbm.at[idxv], sem).wait()
        pltpu.sync_copy(g16_hbm.at[k, wid], g16v)
        pltpu.async_copy(g16v, gs_hbm.at[idxv], sem).wait()


def _ffn_body(meta_ref, xs_ref, w1_ref, b1_ref, w2_ref, b2_ref, gs_ref,
              out_ref):
    m = pl.program_id(0)

    @pl.when(meta_ref[1, m] == 1)
    def _():
        xt = xs_ref[...].astype(jnp.bfloat16)
        w1 = w1_ref[...].astype(jnp.bfloat16)
        h = jnp.dot(xt, w1, preferred_element_type=jnp.float32)
        h = jnp.maximum(h + b1_ref[0][None, :], 0.0).astype(jnp.bfloat16)
        w2 = w2_ref[...].astype(jnp.bfloat16)
        eo = jnp.dot(h, w2, preferred_element_type=jnp.float32)
        eo = eo + b2_ref[0][None, :]
        out_ref[...] = eo * gs_ref[...][:, 0:1]


CH = 32


def _combine_body(eo_hbm, pos_hbm, out_hbm, av, bv, idx0v, idx1v, sem0, sem1):
    wid = lax.axis_index("s") * NC + lax.axis_index("c")
    base = wid * TPW
    pltpu.sync_copy(pos_hbm.at[0, wid], idx0v)
    pltpu.sync_copy(pos_hbm.at[1, wid], idx1v)
    for h in range(TPW // CH):
        c0 = pltpu.async_copy(
            eo_hbm.at[idx0v.at[pl.ds(CH * h, CH)]], av, sem0)
        c1 = pltpu.async_copy(
            eo_hbm.at[idx1v.at[pl.ds(CH * h, CH)]], bv, sem1)
        c0.wait()
        c1.wait()
        for r in range(CH):
            def _add(w, carry):
                sl = pl.ds(16 * w, 16)
                av[r, sl] = av[r, sl] + bv[r, sl]
                return carry
            lax.fori_loop(0, D // 16, _add, 0)
        pltpu.sync_copy(av, out_hbm.at[pl.ds(base + CH * h, CH)])


@jax.jit
def _moe(x, gate_W, gate_b, W1, b1, W2, b2):
    probs, tki, pos, gates, g16, meta = pl.pallas_call(
        _gating_body,
        out_shape=(
            jax.ShapeDtypeStruct((B, E), jnp.float32),
            jax.ShapeDtypeStruct((B, K), jnp.int32),
            jax.ShapeDtypeStruct((B, K), jnp.int32),
            jax.ShapeDtypeStruct((B, K), jnp.float32),
            jax.ShapeDtypeStruct((B, K * 128), jnp.float32),
            jax.ShapeDtypeStruct((4, 128), jnp.int32),
        ),
    )(x, gate_W, gate_b)

    pos_kw = pos.T.reshape(K, NW, TPW)
    g16_kw = g16.reshape(B, K, 128).transpose(1, 0, 2).reshape(K, NW, TPW, 128)

    mesh = plsc.VectorSubcoreMesh(core_axis_name="c", subcore_axis_name="s")
    xs, gs = pl.kernel(
        _dispatch_body,
        out_type=(
            jax.ShapeDtypeStruct((P, D), jnp.float32),
            jax.ShapeDtypeStruct((P, 128), jnp.float32),
        ),
        mesh=mesh,
        scratch_types=[
            pltpu.VMEM((TPW, D), jnp.float32),
            pltpu.VMEM((TPW,), jnp.int32),
            pltpu.VMEM((TPW, 128), jnp.float32),
            pltpu.SemaphoreType.DMA,
        ],
    )(x, pos_kw, g16_kw)

    eo = pl.pallas_call(
        _ffn_body,
        grid_spec=pltpu.PrefetchScalarGridSpec(
            num_scalar_prefetch=1,
            grid=(MT,),
            in_specs=[
                pl.BlockSpec((T, D), lambda m, meta: (meta[2, m], 0)),
                pl.BlockSpec((None, D, H), lambda m, meta: (meta[0, m], 0, 0)),
                pl.BlockSpec((None, 1, H), lambda m, meta: (meta[0, m], 0, 0)),
                pl.BlockSpec((None, H, D), lambda m, meta: (meta[0, m], 0, 0)),
                pl.BlockSpec((None, 1, D), lambda m, meta: (meta[0, m], 0, 0)),
                pl.BlockSpec((T, 128), lambda m, meta: (meta[2, m], 0)),
            ],
            out_specs=pl.BlockSpec((T, D), lambda m, meta: (meta[2, m], 0)),
        ),
        out_shape=jax.ShapeDtypeStruct((P, D), jnp.float32),
    )(meta, xs, W1, b1[:, None, :], W2, b2[:, None, :], gs)

    out = pl.kernel(
        _combine_body,
        out_type=jax.ShapeDtypeStruct((B, D), jnp.float32),
        mesh=mesh,
        scratch_types=[
            pltpu.VMEM((CH, D), jnp.float32),
            pltpu.VMEM((CH, D), jnp.float32),
            pltpu.VMEM((TPW,), jnp.int32),
            pltpu.VMEM((TPW,), jnp.int32),
            pltpu.SemaphoreType.DMA,
            pltpu.SemaphoreType.DMA,
        ],
    )(eo, pos_kw)

    return out, probs, tki


def kernel(x, gate_W, gate_b, W1, b1, W2, b2):
    return _moe(x, gate_W, gate_b, W1, b1, W2, b2)


# speedup vs baseline: 1.1412x; 1.1276x over previous
import functools

import jax
import jax.numpy as jnp
from jax import lax
from jax.experimental import pallas as pl
from jax.experimental.pallas import tpu as pltpu
from jax.experimental.pallas import tpu_sc as plsc

E = 8
K = 2
D = 1024
H = 1024
B = 2048

T = 256
MT = 24
P = MT * T
NC = 2
NS = 16
NW = NC * NS
TPW = B // NW
NEG = -1e30


def _gating_body(x_ref, gw_ref, gb_ref, probs_ref, tki_ref, pos_ref, g_ref,
                 g16_ref, meta_ref):
    x = x_ref[...]
    logits = jnp.dot(x, gw_ref[...], preferred_element_type=jnp.float32)
    logits = logits + gb_ref[...][None, :]
    m = jnp.max(logits, axis=1, keepdims=True)
    ex = jnp.exp(logits - m)
    probs_ref[...] = ex / jnp.sum(ex, axis=1, keepdims=True)

    idx = lax.broadcasted_iota(jnp.int32, (B, E), 1)
    i1 = jnp.min(jnp.where(logits == m, idx, E), axis=1, keepdims=True)
    oh1 = idx == i1
    masked = jnp.where(oh1, NEG, logits)
    m2 = jnp.max(masked, axis=1, keepdims=True)
    i2 = jnp.min(jnp.where(masked == m2, idx, E), axis=1, keepdims=True)
    oh2 = idx == i2
    tki_ref[...] = jnp.concatenate([i1, i2], axis=1)

    b = jnp.exp(m2 - m)
    g1 = 1.0 / (1.0 + b)
    g2 = b / (1.0 + b)
    g_ref[...] = jnp.concatenate([g1, g2], axis=1)
    ones16 = jnp.ones((1, 128), jnp.float32)
    g16_ref[...] = jnp.concatenate([g1 * ones16, g2 * ones16], axis=1)

    cnt = oh1.astype(jnp.int32) + oh2.astype(jnp.int32)
    c = cnt
    s = 1
    while s < B:
        c = c + jnp.concatenate(
            [jnp.zeros((s, E), jnp.int32), c[:-s]], axis=0)
        s *= 2
    counts = c[B - 1:B, :]
    tiles_e = (counts + (T - 1)) // T
    tin = tiles_e
    s = 1
    while s < E:
        tin = tin + jnp.concatenate(
            [jnp.zeros((1, s), jnp.int32), tin[:, :-s]], axis=1)
        s *= 2
    toff = tin - tiles_e
    num_tiles = tin[:, E - 1:E]

    rank1 = jnp.sum(c * oh1, axis=1, keepdims=True) - 1
    rank2 = jnp.sum(c * oh2, axis=1, keepdims=True) - 1
    base1 = jnp.sum(jnp.broadcast_to(toff, (B, E)) * oh1, axis=1,
                    keepdims=True) * T
    base2 = jnp.sum(jnp.broadcast_to(toff, (B, E)) * oh2, axis=1,
                    keepdims=True) * T
    pos_ref[...] = jnp.concatenate([base1 + rank1, base2 + rank2], axis=1)

    miota = lax.broadcasted_iota(jnp.int32, (1, 128), 1)
    eot = jnp.zeros((1, 128), jnp.int32)
    for e in range(E):
        eot = eot + (miota >= toff[0, e]).astype(jnp.int32)
    eot = eot - 1
    real = (miota < num_tiles).astype(jnp.int32)
    xidx = jnp.minimum(miota, num_tiles - 1)
    meta_ref[...] = jnp.concatenate(
        [eot, real, xidx, jnp.zeros((1, 128), jnp.int32)], axis=0)


def _dispatch_body(x_hbm, pos_hbm, g16_hbm, xs_hbm, gs_hbm,
                   xv, idx0v, idx1v, g16v0, g16v1, sem):
    wid = lax.axis_index("s") * NC + lax.axis_index("c")
    base = wid * TPW
    pltpu.sync_copy(pos_hbm.at[0, wid], idx0v)
    pltpu.sync_copy(pos_hbm.at[1, wid], idx1v)
    pltpu.sync_copy(g16_hbm.at[0, wid], g16v0)
    pltpu.sync_copy(g16_hbm.at[1, wid], g16v1)
    pltpu.sync_copy(x_hbm.at[pl.ds(base, TPW)], xv)
    c0 = pltpu.async_copy(xv, xs_hbm.at[idx0v], sem)
    c1 = pltpu.async_copy(xv, xs_hbm.at[idx1v], sem)
    c2 = pltpu.async_copy(g16v0, gs_hbm.at[idx0v], sem)
    c3 = pltpu.async_copy(g16v1, gs_hbm.at[idx1v], sem)
    c0.wait()
    c1.wait()
    c2.wait()
    c3.wait()


def _ffn_body(meta_ref, xs_ref, w1_ref, b1_ref, w2_ref, b2_ref, gs_ref,
              out_ref):
    m = pl.program_id(0)

    @pl.when(meta_ref[1, m] == 1)
    def _():
        xt = xs_ref[...].astype(jnp.bfloat16)
        w1 = w1_ref[...].astype(jnp.bfloat16)
        h = jnp.dot(xt, w1, preferred_element_type=jnp.float32)
        h = jnp.maximum(h + b1_ref[0][None, :], 0.0).astype(jnp.bfloat16)
        w2 = w2_ref[...].astype(jnp.bfloat16)
        eo = jnp.dot(h, w2, preferred_element_type=jnp.float32)
        eo = eo + b2_ref[0][None, :]
        out_ref[...] = eo * gs_ref[...][:, 0:1]


CH = 32


def _combine_body(eo_hbm, pos_hbm, out_hbm, av, bv, idx0v, idx1v, sem0, sem1):
    wid = lax.axis_index("s") * NC + lax.axis_index("c")
    base = wid * TPW
    pltpu.sync_copy(pos_hbm.at[0, wid], idx0v)
    pltpu.sync_copy(pos_hbm.at[1, wid], idx1v)
    for h in range(TPW // CH):
        c0 = pltpu.async_copy(
            eo_hbm.at[idx0v.at[pl.ds(CH * h, CH)]], av, sem0)
        c1 = pltpu.async_copy(
            eo_hbm.at[idx1v.at[pl.ds(CH * h, CH)]], bv, sem1)
        c0.wait()
        c1.wait()
        nsl = D // 16

        @plsc.parallel_loop(0, CH * nsl, unroll=8)
        def _add(j):
            r = j // nsl
            sl = pl.ds(16 * (j % nsl), 16)
            av[r, sl] = av[r, sl] + bv[r, sl]

        pltpu.sync_copy(av, out_hbm.at[pl.ds(base + CH * h, CH)])


@jax.jit
def _moe(x, gate_W, gate_b, W1, b1, W2, b2):
    probs, tki, pos, gates, g16, meta = pl.pallas_call(
        _gating_body,
        out_shape=(
            jax.ShapeDtypeStruct((B, E), jnp.float32),
            jax.ShapeDtypeStruct((B, K), jnp.int32),
            jax.ShapeDtypeStruct((B, K), jnp.int32),
            jax.ShapeDtypeStruct((B, K), jnp.float32),
            jax.ShapeDtypeStruct((B, K * 128), jnp.float32),
            jax.ShapeDtypeStruct((4, 128), jnp.int32),
        ),
    )(x, gate_W, gate_b)

    pos_kw = pos.T.reshape(K, NW, TPW)
    g16_kw = g16.reshape(B, K, 128).transpose(1, 0, 2).reshape(K, NW, TPW, 128)

    mesh = plsc.VectorSubcoreMesh(core_axis_name="c", subcore_axis_name="s")
    xs, gs = pl.kernel(
        _dispatch_body,
        out_type=(
            jax.ShapeDtypeStruct((P, D), jnp.float32),
            jax.ShapeDtypeStruct((P, 128), jnp.float32),
        ),
        mesh=mesh,
        scratch_types=[
            pltpu.VMEM((TPW, D), jnp.float32),
            pltpu.VMEM((TPW,), jnp.int32),
            pltpu.VMEM((TPW,), jnp.int32),
            pltpu.VMEM((TPW, 128), jnp.float32),
            pltpu.VMEM((TPW, 128), jnp.float32),
            pltpu.SemaphoreType.DMA,
        ],
    )(x, pos_kw, g16_kw)

    eo = pl.pallas_call(
        _ffn_body,
        grid_spec=pltpu.PrefetchScalarGridSpec(
            num_scalar_prefetch=1,
            grid=(MT,),
            in_specs=[
                pl.BlockSpec((T, D), lambda m, meta: (meta[2, m], 0)),
                pl.BlockSpec((None, D, H), lambda m, meta: (meta[0, m], 0, 0)),
                pl.BlockSpec((None, 1, H), lambda m, meta: (meta[0, m], 0, 0)),
                pl.BlockSpec((None, H, D), lambda m, meta: (meta[0, m], 0, 0)),
                pl.BlockSpec((None, 1, D), lambda m, meta: (meta[0, m], 0, 0)),
                pl.BlockSpec((T, 128), lambda m, meta: (meta[2, m], 0)),
            ],
            out_specs=pl.BlockSpec((T, D), lambda m, meta: (meta[2, m], 0)),
        ),
        out_shape=jax.ShapeDtypeStruct((P, D), jnp.float32),
    )(meta, xs, W1, b1[:, None, :], W2, b2[:, None, :], gs)

    out = pl.kernel(
        _combine_body,
        out_type=jax.ShapeDtypeStruct((B, D), jnp.float32),
        mesh=mesh,
        scratch_types=[
            pltpu.VMEM((CH, D), jnp.float32),
            pltpu.VMEM((CH, D), jnp.float32),
            pltpu.VMEM((TPW,), jnp.int32),
            pltpu.VMEM((TPW,), jnp.int32),
            pltpu.SemaphoreType.DMA,
            pltpu.SemaphoreType.DMA,
        ],
    )(eo, pos_kw)

    return out, probs, tki


def kernel(x, gate_W, gate_b, W1, b1, W2, b2):
    return _moe(x, gate_W, gate_b, W1, b1, W2, b2)
